# Initial kernel scaffold; baseline (speedup 1.0000x reference)
import jax
import jax.numpy as jnp
from jax.experimental import pallas as pl


def kernel(embed, objs, obj_vecs, rels, pred_vecs, edges, rel_word_nums, W1a, b1a, W1b, b1b, W2a, b2a, W2b, b2b):
    raise NotImplementedError("write your pallas kernel here")


# speedup vs baseline: 3.7275x; 3.7275x over previous
import functools

import jax
import jax.numpy as jnp
from jax import lax
from jax.experimental import pallas as pl
from jax.experimental.pallas import tpu as pltpu
from jax.experimental.pallas import tpu_sc as plsc

F32 = jnp.float32
NC = 2
NS = 16
NW = NC * NS


def _leaky(x):
    return jnp.where(x > 0, x, 0.01 * x)


def _premul_body(ov_ref, w1a_ref, u_ref, v_ref):
    d = ov_ref.shape[1]
    ov = ov_ref[...]
    u_ref[...] = jnp.dot(ov, w1a_ref[0:d, :], preferred_element_type=F32)
    v_ref[...] = jnp.dot(ov, w1a_ref[2 * d:3 * d, :], preferred_element_type=F32)


def _edge_body(gs_ref, go_ref, p_ref, wp_ref, b1a_ref, w1b_ref, b1b_ref,
               np_ref, ns_ref, no_ref):
    h = w1b_ref.shape[0]
    bf = jnp.bfloat16
    h1 = (gs_ref[...] + go_ref[...]
          + jnp.dot(p_ref[...].astype(bf), wp_ref[...].astype(bf),
                    preferred_element_type=F32)
          + b1a_ref[...])
    h1 = _leaky(h1)
    t = (jnp.dot(h1.astype(bf), w1b_ref[...].astype(bf),
                 preferred_element_type=F32) + b1b_ref[...])
    t = _leaky(t)
    ns_ref[...] = t[:, 0:h]
    np_ref[...] = t[:, h:2 * h]
    no_ref[...] = t[:, 2 * h:3 * h]


def _node_body(p2_ref, c2_ref, w2a_ref, b2a_ref, w2b_ref, b2b_ref, out_ref):
    n = out_ref.shape[0]
    h = n // 2
    opadh = p2_ref.shape[1]
    pooled = jnp.concatenate([p2_ref[0, 0:h, :], p2_ref[1, 0:h, :]], axis=0)
    cnt = jnp.concatenate([c2_ref[0:h], c2_ref[opadh:opadh + h]], axis=0)
    pooled = pooled / jnp.maximum(cnt, 1.0).reshape(n, 1)
    h2 = _leaky(jnp.dot(pooled, w2a_ref[...], preferred_element_type=F32)
                + b2a_ref[...])
    out_ref[...] = _leaky(jnp.dot(h2, w2b_ref[...], preferred_element_type=F32)
                          + b2b_ref[...])


def _make_sc_gather(T, O, D, CH):
    TW = T // NW
    NCH = TW // CH
    K = NCH // 2
    mesh = plsc.VectorSubcoreMesh(core_axis_name="c", subcore_axis_name="s")

    @functools.partial(
        pl.kernel,
        out_type=(jax.ShapeDtypeStruct((T, D), F32),
                  jax.ShapeDtypeStruct((T, D), F32)),
        mesh=mesh,
        scratch_types=[
            [pltpu.VMEM((CH,), jnp.int32)] * 2,
            [pltpu.VMEM((CH,), jnp.int32)] * 2,
            [pltpu.VMEM((CH, D), F32)] * 2,
            [pltpu.VMEM((CH, D), F32)] * 2,
            [pltpu.SemaphoreType.DMA] * 2,
            [pltpu.SemaphoreType.DMA] * 2,
            [pltpu.SemaphoreType.DMA] * 2,
        ],
    )
    def sc_gather(s_hbm, o_hbm, u_hbm, v_hbm, gs_hbm, go_hbm,
                  sidxs, oidxs, srowss, orowss, isems, gsems, wsems):
        wid = lax.axis_index("s") * NC + lax.axis_index("c")
        base = wid * TW

        def chunk_pair(i0):
            di = []
            for u in range(2):
                off = base + (i0 + u) * CH
                di.append((
                    pltpu.async_copy(s_hbm.at[pl.ds(off, CH)], sidxs[u],
                                     isems[u]),
                    pltpu.async_copy(o_hbm.at[pl.ds(off, CH)], oidxs[u],
                                     isems[u]),
                ))
            dg = []
            for u in range(2):
                di[u][0].wait()
                di[u][1].wait()
                dg.append((
                    pltpu.async_copy(u_hbm.at[sidxs[u]], srowss[u], gsems[u]),
                    pltpu.async_copy(v_hbm.at[oidxs[u]], orowss[u], gsems[u]),
                ))
            dw = []
            for u in range(2):
                dg[u][0].wait()
                dg[u][1].wait()
                off = base + (i0 + u) * CH
                dw.append((
                    pltpu.async_copy(srowss[u], gs_hbm.at[pl.ds(off, CH)],
                                     wsems[u]),
                    pltpu.async_copy(orowss[u], go_hbm.at[pl.ds(off, CH)],
                                     wsems[u]),
                ))
            for u in range(2):
                dw[u][0].wait()
                dw[u][1].wait()

        def body(k, carry):
            chunk_pair(2 * k)
            return carry

        lax.fori_loop(0, K, body, 0)
        if NCH % 2:
            off = base + (NCH - 1) * CH
            pltpu.sync_copy(s_hbm.at[pl.ds(off, CH)], sidxs[0])
            pltpu.sync_copy(o_hbm.at[pl.ds(off, CH)], oidxs[0])
            g1 = pltpu.async_copy(u_hbm.at[sidxs[0]], srowss[0], gsems[0])
            g2 = pltpu.async_copy(v_hbm.at[oidxs[0]], orowss[0], gsems[0])
            g1.wait()
            g2.wait()
            pltpu.sync_copy(srowss[0], gs_hbm.at[pl.ds(off, CH)])
            pltpu.sync_copy(orowss[0], go_hbm.at[pl.ds(off, CH)])

    return sc_gather


def _make_sc_scatter(TH, OH, D, CH, phase):
    TW = TH // NS
    NCH = TW // CH
    K = NCH // 2
    assert CH % 16 == 0 and TW % CH == 0
    RPT = ((OH + NS * 16 + NS - 1) // NS + 335) // 336 * 336
    ZR = 56
    OPADH = NS * RPT

    mesh = plsc.VectorSubcoreMesh(core_axis_name="c", subcore_axis_name="s")
    if phase < 2:
        outs = (jax.ShapeDtypeStruct((NC, OPADH, D), F32),
                jax.ShapeDtypeStruct((NC * NS * OPADH,), F32))
    else:
        outs = (jax.ShapeDtypeStruct((NC, OPADH, D), F32),
                jax.ShapeDtypeStruct((NC * OPADH,), F32))

    @functools.partial(
        pl.kernel,
        out_type=outs,
        mesh=mesh,
        compiler_params=pltpu.CompilerParams(needs_layout_passes=False),
        scratch_types=[
            [pltpu.VMEM((CH,), jnp.int32)] * 4,
            [pltpu.VMEM((CH, D), F32)] * 4,
            [pltpu.SemaphoreType.DMA] * 4,
            [pltpu.SemaphoreType.DMA] * 4,
            pltpu.VMEM((ZR, D), F32),
            pltpu.VMEM((OPADH,), F32),
            pltpu.VMEM((RPT,), F32),
            pltpu.VMEM((RPT,), F32),
            pltpu.VMEM_SHARED((OPADH, D), F32),
            pltpu.VMEM_SHARED((NS * OPADH,), F32),
        ],
    )
    def sc_scatter(s_hbm, o_hbm, ns_hbm, no_hbm, *rest):
        if phase == 0:
            (pooled_hbm, out2,
             ibs, rbs, lsems, csems, zrow_v, hist_v, acc_v, tmp_v,
             pooled_sh, hist_sh) = rest
        else:
            (pooled_in, hist_in, pooled_hbm, out2,
             ibs, rbs, lsems, csems, zrow_v, hist_v, acc_v, tmp_v,
             pooled_sh, hist_sh) = rest
        cid = lax.axis_index("c")
        sid = lax.axis_index("s")
        base = sid * TW
        half_lo = cid * OH
        dump = OH + sid * 16 + lax.iota(jnp.int32, 16)
        ones16 = jnp.ones((16,), F32)

        idx_hbms = (s_hbm, o_hbm)
        row_hbms = (ns_hbm, no_hbm)
        tb = sid * RPT

        if phase == 0:
            def fill_zrow(r, carry):
                for c in range(D // 16):
                    zrow_v[r, pl.ds(c * 16, 16)] = jnp.zeros((16,), F32)
                return carry
            lax.fori_loop(0, ZR, fill_zrow, 0)

            def fill_hist(r, carry):
                hist_v[pl.ds(r * 16, 16)] = jnp.zeros((16,), F32)
                return carry
            lax.fori_loop(0, OPADH // 16, fill_hist, 0)

            for k in range(RPT // ZR):
                pltpu.sync_copy(zrow_v, pooled_sh.at[pl.ds(tb + k * ZR, ZR)])
        else:
            for k in range(RPT // ZR):
                pltpu.sync_copy(pooled_in.at[cid, pl.ds(tb + k * ZR, ZR)],
                                zrow_v)
                pltpu.sync_copy(zrow_v, pooled_sh.at[pl.ds(tb + k * ZR, ZR)])
            pltpu.sync_copy(
                hist_in.at[pl.ds((cid * NS + sid) * OPADH, OPADH)], hist_v)
        plsc.subcore_barrier()

        def remap_and_count(ib):
            for j in range(CH // 16):
                ix = ib[pl.ds(j * 16, 16)]
                loc = ix - half_lo
                ok = (loc >= 0) & (loc < OH)
                mapped = jnp.where(ok, loc, dump)
                ib[pl.ds(j * 16, 16)] = mapped
                plsc.addupdate_scatter(hist_v, [mapped], ones16)

        def issue_loads(p, i):
            w = p % 2
            d1 = pltpu.async_copy(idx_hbms[w].at[pl.ds(base + i * CH, CH)],
                                  ibs[p], lsems[p])
            d2 = pltpu.async_copy(row_hbms[w].at[pl.ds(base + i * CH, CH)],
                                  rbs[p], lsems[p])
            return d1, d2

        def body(k, carry):
            descs = [issue_loads(p, 2 * k + p // 2) for p in range(4)]
            sdescs = []
            for p in range(4):
                d1, d2 = descs[p]
                d1.wait()
                d2.wait()
                remap_and_count(ibs[p])
                sdescs.append(pltpu.async_copy(rbs[p], pooled_sh.at[ibs[p]],
                                               csems[p], add=True))
            for d in sdescs:
                d.wait()
            return carry

        lax.fori_loop(0, K, body, 0)
        if NCH % 2:
            for w in range(2):
                off = base + (NCH - 1) * CH
                pltpu.sync_copy(idx_hbms[w].at[pl.ds(off, CH)], ibs[0])
                pltpu.sync_copy(row_hbms[w].at[pl.ds(off, CH)], rbs[0])
                remap_and_count(ibs[0])
                pltpu.sync_copy(rbs[0], pooled_sh.at[ibs[0]], add=True)
        plsc.subcore_barrier()

        for k in range(RPT // ZR):
            pltpu.sync_copy(pooled_sh.at[pl.ds(tb + k * ZR, ZR)], zrow_v)
            pltpu.sync_copy(zrow_v, pooled_hbm.at[cid, pl.ds(tb + k * ZR, ZR)])

        if phase < 2:
            pltpu.sync_copy(
                hist_v, out2.at[pl.ds((cid * NS + sid) * OPADH, OPADH)])
        else:
            pltpu.sync_copy(hist_v, hist_sh.at[pl.ds(sid * OPADH, OPADH)])
            plsc.subcore_barrier()

            def zero_acc(r, carry):
                acc_v[pl.ds(r * 16, 16)] = jnp.zeros((16,), F32)
                return carry
            lax.fori_loop(0, RPT // 16, zero_acc, 0)
            for t in range(NS):
                pltpu.sync_copy(hist_sh.at[pl.ds(t * OPADH + tb, RPT)], tmp_v)

                def add_tmp(r, carry):
                    acc_v[pl.ds(r * 16, 16)] = (acc_v[pl.ds(r * 16, 16)]
                                                + tmp_v[pl.ds(r * 16, 16)])
                    return carry
                lax.fori_loop(0, RPT // 16, add_tmp, 0)
            pltpu.sync_copy(acc_v,
                            out2.at[pl.ds(cid * OPADH + tb, RPT)])

    return sc_scatter


def kernel(embed, objs, obj_vecs, rels, pred_vecs, edges, rel_word_nums,
           W1a, b1a, W1b, b1b, W2a, b2a, W2b, b2b):
    bs, O, Din = obj_vecs.shape
    T = pred_vecs.shape[1]
    H = W1a.shape[1]
    Dout = W1b.shape[1] - 2 * H

    s_idx = edges[0, :, 0].astype(jnp.int32)
    o_idx = edges[0, :, 1].astype(jnp.int32)
    ov = obj_vecs.reshape(O, Din)
    pred = pred_vecs.reshape(T, Din)

    u, v = pl.pallas_call(
        _premul_body,
        out_shape=(jax.ShapeDtypeStruct((O, H), F32),
                   jax.ShapeDtypeStruct((O, H), F32)),
    )(ov, W1a)

    BT = 640
    row_spec = pl.BlockSpec((BT, H), lambda i: (i, 0))
    full = lambda shape: pl.BlockSpec(shape, lambda i: (0,) * len(shape))

    def edge_chunk(gs, go, pred_h, tl):
        return pl.pallas_call(
            _edge_body,
            grid=(tl // BT,),
            in_specs=[row_spec, row_spec, row_spec,
                      full((Din, H)), full((1, H)),
                      full((H, 2 * H + Dout)), full((1, 2 * H + Dout))],
            out_specs=[row_spec, row_spec, row_spec],
            out_shape=(jax.ShapeDtypeStruct((tl, Dout), F32),
                       jax.ShapeDtypeStruct((tl, H), F32),
                       jax.ShapeDtypeStruct((tl, H), F32)),
        )(gs, go, pred_h, W1a[Din:2 * Din, :], b1a.reshape(1, H),
          W1b, b1b.reshape(1, 2 * H + Dout))

    bounds = [(i * T // 5, T // 5) for i in range(5)]
    gathered = []
    for lo, tl in bounds:
        gathered.append(_make_sc_gather(tl, O, H, 40)(
            s_idx[lo:lo + tl], o_idx[lo:lo + tl], u, v))
    edged = []
    for (lo, tl), (gs, go) in zip(bounds, gathered):
        edged.append(edge_chunk(gs, go, pred[lo:lo + tl], tl))

    state = None
    for ci, ((lo, tl), (np_c, ns_c, no_c)) in enumerate(zip(bounds, edged)):
        phase = 0 if ci == 0 else (2 if ci == len(bounds) - 1 else 1)
        args = (s_idx[lo:lo + tl], o_idx[lo:lo + tl], ns_c, no_c)
        if state is not None:
            args = args + state
        state = _make_sc_scatter(tl, O // 2, H, 80, phase)(*args)
    pooled2, counts2 = state

    new_p = jnp.concatenate([e[0] for e in edged], axis=0)

    new_obj = pl.pallas_call(
        _node_body,
        out_shape=jax.ShapeDtypeStruct((O, Dout), F32),
    )(pooled2, counts2, W2a, b2a.reshape(1, H), W2b, b2b.reshape(1, Dout))

    return (new_obj.reshape(bs, O, Dout), new_p.reshape(bs, T, Dout))
